# baseline (speedup 1.0000x reference)
import jax
import jax.numpy as jnp
from jax import lax
from jax.experimental import pallas as pl
from jax.experimental.pallas import tpu as pltpu
from jax.experimental.pallas import tpu_sc as plsc

N_NODES = 100000
D = 32
E = 1280000

NC, NS = 2, 16
W = NC * NS
HW = W // 2
EW = (E // 2) // HW
GE = 320
NG = EW // GE
NST = 2 * GE // 128

FOLD = 4
RB = 1000


def _normalize(emb128):
    def body(x_ref, o_ref):
        x = x_ref[...]
        r = lax.broadcasted_iota(jnp.int32, (FOLD * D, FOLD * D), 0) // D
        c = lax.broadcasted_iota(jnp.int32, (FOLD * D, FOLD * D), 1) // D
        m = (r == c).astype(jnp.float32)
        s = lax.dot_general(x * x, m, (((1,), (0,)), ((), ())),
                            preferred_element_type=jnp.float32)
        o_ref[...] = x / jnp.maximum(jnp.sqrt(s), 1e-12)

    n128 = N_NODES // FOLD
    return pl.pallas_call(
        body,
        grid=(n128 // RB,),
        in_specs=[pl.BlockSpec((RB, FOLD * D), lambda i: (i, 0))],
        out_specs=pl.BlockSpec((RB, FOLD * D), lambda i: (i, 0)),
        out_shape=jax.ShapeDtypeStruct((n128, FOLD * D), jnp.float32),
    )(emb128)


def _run(table, eflat, out, idx_v, buf, out_v, sem, isem, fbase, obase):
    def idx_copy(g, slot):
        pltpu.async_copy(eflat.at[pl.ds(fbase + g * 2 * GE, 2 * GE)],
                         idx_v.at[slot], isem.at[slot])

    def idx_wait(slot):
        pltpu.make_async_copy(eflat.at[pl.ds(0, 2 * GE)], idx_v.at[slot],
                              isem.at[slot]).wait()

    def fire(slot, b):
        for j in range(NST):
            pltpu.async_copy(table.at[idx_v.at[slot, pl.ds(j * 128, 128)]],
                             buf.at[b, pl.ds(j * 128, 128)], sem.at[b])

    idx_copy(0, 0)
    idx_copy(1, 1)
    idx_wait(0)
    fire(0, 0)
    lanes = lax.iota(jnp.int32, 16)

    def group_body(g, carry):
        cur = lax.rem(g, 2)
        nxt = 1 - cur

        @pl.when(g < NG - 1)
        def _():
            idx_wait(lax.rem(g + 1, 3))
            fire(lax.rem(g + 1, 3), nxt)

        @pl.when(g < NG - 2)
        def _():
            idx_copy(g + 2, lax.rem(g + 2, 3))

        for j in range(NST):
            pltpu.make_async_copy(table.at[pl.ds(0, 128)],
                                  buf.at[cur, pl.ds(0, 128)],
                                  sem.at[cur]).wait()

        bref = buf.at[cur]

        def batch_body(b, c):
            row2 = 2 * (lanes + 16 * b)
            acc = jnp.zeros((16,), jnp.float32)
            for d in range(D):
                col = (lanes + d) & 31
                sv = plsc.load_gather(bref, [row2, col])
                dv = plsc.load_gather(bref, [row2 + 1, col])
                acc = acc + sv * dv
            out_v[cur, pl.ds(16 * b, 16)] = acc
            return c

        lax.fori_loop(0, GE // 16, batch_body, 0)
        pltpu.sync_copy(out_v.at[cur], out.at[pl.ds(obase + g * GE, GE)])
        return carry

    lax.fori_loop(0, NG, group_body, 0)


def _sc_body(table, pos_flat, neg_flat, out, idx_v, buf, out_v, sem, isem):
    wid = lax.axis_index("s") * NC + lax.axis_index("c")

    @pl.when(wid < HW)
    def _():
        _run(table, pos_flat, out, idx_v, buf, out_v, sem, isem,
             wid * 2 * EW, wid * EW)

    @pl.when(wid >= HW)
    def _():
        hw = wid - HW
        _run(table, neg_flat, out, idx_v, buf, out_v, sem, isem,
             hw * 2 * EW, E // 2 + hw * EW)


def kernel(embeddings, pos_edges, neg_edges):
    emb_n = _normalize(embeddings.reshape(N_NODES // FOLD, FOLD * D))
    sc = pl.kernel(
        _sc_body,
        out_type=jax.ShapeDtypeStruct((E,), jnp.float32),
        mesh=plsc.VectorSubcoreMesh(core_axis_name="c", subcore_axis_name="s"),
        scratch_types=[
            pltpu.VMEM((3, 2 * GE), jnp.int32),
            pltpu.VMEM((2, 2 * GE, D), jnp.float32),
            pltpu.VMEM((2, GE), jnp.float32),
            pltpu.SemaphoreType.DMA((2,)),
            pltpu.SemaphoreType.DMA((3,)),
        ],
        compiler_params=pltpu.CompilerParams(
            needs_layout_passes=False, use_tc_tiling_on_sc=False),
    )
    return sc(emb_n.reshape(N_NODES, D),
              pos_edges.reshape(-1), neg_edges.reshape(-1))


# speedup vs baseline: 3.2596x; 3.2596x over previous
import jax
import jax.numpy as jnp
from jax import lax
from jax.experimental import pallas as pl
from jax.experimental.pallas import tpu as pltpu
from jax.experimental.pallas import tpu_sc as plsc

N_NODES = 100000
D = 32
E = 1280000

NC, NS = 2, 16
W = NC * NS
EW = E // W
GE = 320
NG = EW // GE
SPG = 4
SR = GE // SPG

NR = 3128
NGR = 10

_LANES = None


def _rsqrt(ss):
    ii = lax.bitcast_convert_type(ss, jnp.int32)
    ii = jnp.int32(0x5F3759DF) - (ii >> 1)
    y = lax.bitcast_convert_type(ii, jnp.float32)
    for _ in range(3):
        y = y * (1.5 - 0.5 * ss * y * y)
    return jnp.minimum(y, 1e12)


def _sc_norm_body(table, out, buf, rs_v, isem):
    wid = lax.axis_index("s") * NC + lax.axis_index("c")
    base = wid * NR
    span = jnp.minimum(NR, N_NODES - base)
    lanes = lax.iota(jnp.int32, 16)

    def row0(g):
        return base + jnp.minimum(g * GE, span - GE)

    def copy_in(g, slot):
        pltpu.async_copy(table.at[pl.ds(row0(g), GE)], buf.at[slot],
                         isem.at[slot])

    def wait_in(slot):
        pltpu.make_async_copy(table.at[pl.ds(0, GE)], buf.at[slot],
                              isem.at[slot]).wait()

    copy_in(0, 0)

    def group_body(g, carry):
        cur = lax.rem(g, 2)

        @pl.when(g < NGR - 1)
        def _():
            copy_in(g + 1, 1 - cur)

        wait_in(cur)
        bref = buf.at[cur]

        def batch_body(b, c):
            row = lanes + 16 * b
            ss = jnp.zeros((16,), jnp.float32)
            for d in range(D):
                col = (lanes + d) & 31
                v = plsc.load_gather(bref, [row, col])
                ss = ss + v * v
            rs_v[cur, pl.ds(16 * b, 16)] = _rsqrt(ss)
            return c

        lax.fori_loop(0, GE // 16, batch_body, 0)
        pltpu.sync_copy(rs_v.at[cur], out.at[pl.ds(row0(g), GE)])
        return carry

    lax.fori_loop(0, NGR, group_body, 0)


def _sc_dot_body(table, rnorm, sidx, didx, out,
                 idx_v, src_v, dst_v, rn_s, rn_d, out_v, sem, nsem, isem):
    wid = lax.axis_index("s") * NC + lax.axis_index("c")
    ebase = wid * EW

    def idx_copy(g, slot):
        base = ebase + g * GE
        pltpu.async_copy(sidx.at[pl.ds(base, GE)], idx_v.at[slot, 0],
                         isem.at[slot])
        pltpu.async_copy(didx.at[pl.ds(base, GE)], idx_v.at[slot, 1],
                         isem.at[slot])

    def idx_wait(slot):
        pltpu.make_async_copy(sidx.at[pl.ds(0, GE)], idx_v.at[slot, 0],
                              isem.at[slot]).wait()
        pltpu.make_async_copy(didx.at[pl.ds(0, GE)], idx_v.at[slot, 1],
                              isem.at[slot]).wait()

    def fire(g, slot, buf):
        for j in range(SPG):
            pltpu.async_copy(table.at[idx_v.at[slot, 0, pl.ds(j * SR, SR)]],
                             src_v.at[buf, pl.ds(j * SR, SR)], sem.at[buf])
            pltpu.async_copy(table.at[idx_v.at[slot, 1, pl.ds(j * SR, SR)]],
                             dst_v.at[buf, pl.ds(j * SR, SR)], sem.at[buf])
        for off, ln in ((0, 128), (128, 128), (256, 64)):
            pltpu.async_copy(rnorm.at[idx_v.at[slot, 0, pl.ds(off, ln)]],
                             rn_s.at[buf, pl.ds(off, ln)], nsem.at[buf])
            pltpu.async_copy(rnorm.at[idx_v.at[slot, 1, pl.ds(off, ln)]],
                             rn_d.at[buf, pl.ds(off, ln)], nsem.at[buf])

    idx_copy(0, 0)
    idx_copy(1, 1)
    idx_wait(0)
    fire(0, 0, 0)
    lanes = lax.iota(jnp.int32, 16)

    def group_body(g, carry):
        cur = lax.rem(g, 2)
        nxt = 1 - cur

        @pl.when(g < NG - 1)
        def _():
            idx_wait(lax.rem(g + 1, 3))
            fire(g + 1, lax.rem(g + 1, 3), nxt)

        @pl.when(g < NG - 2)
        def _():
            idx_copy(g + 2, lax.rem(g + 2, 3))

        for j in range(2 * SPG):
            pltpu.make_async_copy(table.at[pl.ds(0, SR)],
                                  src_v.at[cur, pl.ds(0, SR)],
                                  sem.at[cur]).wait()
        for j in range(4):
            pltpu.make_async_copy(rnorm.at[pl.ds(0, 128)],
                                  rn_s.at[cur, pl.ds(0, 128)],
                                  nsem.at[cur]).wait()
        for j in range(2):
            pltpu.make_async_copy(rnorm.at[pl.ds(0, 64)],
                                  rn_s.at[cur, pl.ds(0, 64)],
                                  nsem.at[cur]).wait()

        sref = src_v.at[cur]
        dref = dst_v.at[cur]

        def batch_body(b, c):
            row = lanes + 16 * b
            acc = jnp.zeros((16,), jnp.float32)
            for d in range(D):
                col = (lanes + d) & 31
                sv = plsc.load_gather(sref, [row, col])
                dv = plsc.load_gather(dref, [row, col])
                acc = acc + sv * dv
            rs = rn_s[cur, pl.ds(16 * b, 16)]
            rd = rn_d[cur, pl.ds(16 * b, 16)]
            out_v[cur, pl.ds(16 * b, 16)] = acc * rs * rd
            return c

        lax.fori_loop(0, GE // 16, batch_body, 0)
        pltpu.sync_copy(out_v.at[cur], out.at[pl.ds(ebase + g * GE, GE)])
        return carry

    lax.fori_loop(0, NG, group_body, 0)


def kernel(embeddings, pos_edges, neg_edges):
    mesh = plsc.VectorSubcoreMesh(core_axis_name="c", subcore_axis_name="s")
    params = pltpu.CompilerParams(
        needs_layout_passes=False, use_tc_tiling_on_sc=False)
    norm = pl.kernel(
        _sc_norm_body,
        out_type=jax.ShapeDtypeStruct((N_NODES,), jnp.float32),
        mesh=mesh,
        scratch_types=[
            pltpu.VMEM((2, GE, D), jnp.float32),
            pltpu.VMEM((2, GE), jnp.float32),
            pltpu.SemaphoreType.DMA((2,)),
        ],
        compiler_params=params,
    )
    rnorm = norm(embeddings)
    sidx = jnp.concatenate([pos_edges[:, 0], neg_edges[:, 0]])
    didx = jnp.concatenate([pos_edges[:, 1], neg_edges[:, 1]])
    dot = pl.kernel(
        _sc_dot_body,
        out_type=jax.ShapeDtypeStruct((E,), jnp.float32),
        mesh=mesh,
        scratch_types=[
            pltpu.VMEM((3, 2, GE), jnp.int32),
            pltpu.VMEM((2, GE, D), jnp.float32),
            pltpu.VMEM((2, GE, D), jnp.float32),
            pltpu.VMEM((2, GE), jnp.float32),
            pltpu.VMEM((2, GE), jnp.float32),
            pltpu.VMEM((2, GE), jnp.float32),
            pltpu.SemaphoreType.DMA((2,)),
            pltpu.SemaphoreType.DMA((2,)),
            pltpu.SemaphoreType.DMA((3,)),
        ],
        compiler_params=params,
    )
    return dot(embeddings, rnorm, sidx, didx)


# speedup vs baseline: 3.9012x; 1.1969x over previous
import jax
import jax.numpy as jnp
from jax import lax
from jax.experimental import pallas as pl
from jax.experimental.pallas import tpu as pltpu
from jax.experimental.pallas import tpu_sc as plsc

N_NODES = 100000
D = 32
E = 1280000

NC, NS = 2, 16
W = NC * NS
EW = E // W
GE = 320
NG = EW // GE
SPG = 4
SR = GE // SPG

FOLD = 4
RB = 1000


def _normalize(emb128):
    def body(x_ref, o_ref):
        x = x_ref[...]
        r = lax.broadcasted_iota(jnp.int32, (FOLD * D, FOLD * D), 0) // D
        c = lax.broadcasted_iota(jnp.int32, (FOLD * D, FOLD * D), 1) // D
        m = (r == c).astype(jnp.float32)
        s = lax.dot_general(x * x, m, (((1,), (0,)), ((), ())),
                            preferred_element_type=jnp.float32)
        o_ref[...] = (x / jnp.maximum(jnp.sqrt(s), 1e-12)).reshape(RB * FOLD * D)

    n128 = N_NODES // FOLD
    return pl.pallas_call(
        body,
        grid=(n128 // RB,),
        in_specs=[pl.BlockSpec((RB, FOLD * D), lambda i: (i, 0))],
        out_specs=pl.BlockSpec((RB * FOLD * D,), lambda i: (i,)),
        out_shape=jax.ShapeDtypeStruct((N_NODES * D,), jnp.float32),
    )(emb128)


def _sc_body(table, sidx, didx, out, idx_v, src_v, dst_v, out_v, sem, isem):
    wid = lax.axis_index("s") * NC + lax.axis_index("c")
    ebase = wid * EW

    def idx_copy(g, slot):
        base = ebase + g * GE
        pltpu.async_copy(sidx.at[pl.ds(base, GE)], idx_v.at[slot, 0],
                         isem.at[slot])
        pltpu.async_copy(didx.at[pl.ds(base, GE)], idx_v.at[slot, 1],
                         isem.at[slot])

    def idx_wait(slot):
        pltpu.make_async_copy(sidx.at[pl.ds(0, GE)], idx_v.at[slot, 0],
                              isem.at[slot]).wait()
        pltpu.make_async_copy(didx.at[pl.ds(0, GE)], idx_v.at[slot, 1],
                              isem.at[slot]).wait()

    def fire(g, slot, buf):
        for j in range(SPG):
            pltpu.async_copy(table.at[idx_v.at[slot, 0, pl.ds(j * SR, SR)]],
                             src_v.at[buf, pl.ds(j * SR, SR)], sem.at[buf])
            pltpu.async_copy(table.at[idx_v.at[slot, 1, pl.ds(j * SR, SR)]],
                             dst_v.at[buf, pl.ds(j * SR, SR)], sem.at[buf])

    idx_copy(0, 0)
    idx_copy(1, 1)
    idx_wait(0)
    fire(0, 0, 0)
    lanes = lax.iota(jnp.int32, 16)

    def group_body(g, carry):
        cur = lax.rem(g, 2)
        nxt = 1 - cur

        @pl.when(g < NG - 1)
        def _():
            idx_wait(lax.rem(g + 1, 3))
            fire(g + 1, lax.rem(g + 1, 3), nxt)

        @pl.when(g < NG - 2)
        def _():
            idx_copy(g + 2, lax.rem(g + 2, 3))

        for j in range(2 * SPG):
            pltpu.make_async_copy(table.at[pl.ds(0, SR)],
                                  src_v.at[cur, pl.ds(0, SR)],
                                  sem.at[cur]).wait()

        sref = src_v.at[cur]
        dref = dst_v.at[cur]

        def batch_body(b, c):
            row = lanes + 16 * b
            acc = jnp.zeros((16,), jnp.float32)
            for d in range(D):
                col = (lanes + d) & 31
                sv = plsc.load_gather(sref, [row, col])
                dv = plsc.load_gather(dref, [row, col])
                acc = acc + sv * dv
            out_v[cur, pl.ds(16 * b, 16)] = acc
            return c

        lax.fori_loop(0, GE // 16, batch_body, 0)
        pltpu.sync_copy(out_v.at[cur], out.at[pl.ds(ebase + g * GE, GE)])
        return carry

    lax.fori_loop(0, NG, group_body, 0)


def kernel(embeddings, pos_edges, neg_edges):
    emb_n = _normalize(embeddings.reshape(N_NODES // FOLD, FOLD * D))
    sidx = jnp.concatenate([pos_edges[:, 0], neg_edges[:, 0]])
    didx = jnp.concatenate([pos_edges[:, 1], neg_edges[:, 1]])
    sc = pl.kernel(
        _sc_body,
        out_type=jax.ShapeDtypeStruct((E,), jnp.float32),
        mesh=plsc.VectorSubcoreMesh(core_axis_name="c", subcore_axis_name="s"),
        scratch_types=[
            pltpu.VMEM((3, 2, GE), jnp.int32),
            pltpu.VMEM((2, GE, D), jnp.float32),
            pltpu.VMEM((2, GE, D), jnp.float32),
            pltpu.VMEM((2, GE), jnp.float32),
            pltpu.SemaphoreType.DMA((2,)),
            pltpu.SemaphoreType.DMA((3,)),
        ],
        compiler_params=pltpu.CompilerParams(
            needs_layout_passes=False, use_tc_tiling_on_sc=False),
    )
    return sc(emb_n.reshape(N_NODES, D), sidx, didx)
